# baseline (speedup 1.0000x reference)
import functools

import jax
import jax.numpy as jnp
from jax import lax
from jax.experimental import pallas as pl
from jax.experimental.pallas import tpu as pltpu
from jax.experimental.pallas import tpu_sc as plsc

_D = 64
_DP = 128
_SP = 56
_NC = 2
_NS = 16
_NW = _NC * _NS
_BB = 4


def _make_gather(n_batch: int):
    assert n_batch % (_NW * _BB) == 0
    bat_per_w = n_batch // _NW
    n_groups = bat_per_w // _BB
    assert n_groups % 2 == 0 and n_groups >= 4
    chunk_rows = _BB * _SP
    mesh = plsc.VectorSubcoreMesh(core_axis_name="c", subcore_axis_name="s")

    @functools.partial(
        pl.kernel,
        out_type=jax.ShapeDtypeStruct((n_batch, _SP, _DP), jnp.float32),
        mesh=mesh,
        scratch_types=[
            pltpu.VMEM((bat_per_w * _SP,), jnp.int32),
            pltpu.VMEM((2, _BB, _SP, _DP), jnp.float32),
            pltpu.SemaphoreType.DMA,
            pltpu.SemaphoreType.DMA,
            pltpu.SemaphoreType.DMA,
        ],
        compiler_params=pltpu.CompilerParams(use_tc_tiling_on_sc=False),
    )
    def gather(table_hbm, idx_hbm, out_hbm, idx_v, rows_v, sem_i, sem_g, sem_w):
        wid = lax.axis_index("s") * _NC + lax.axis_index("c")
        row_base = wid * bat_per_w * _SP
        bat_base = wid * bat_per_w

        def fire_gathers(g, b):
            for i in range(_BB):
                pltpu.async_copy(
                    table_hbm.at[idx_v.at[pl.ds(g * chunk_rows + i * _SP, _SP)]],
                    rows_v.at[b, i], sem_g,
                )

        def drain_gathers(b):
            for i in range(_BB):
                pltpu.make_async_copy(
                    table_hbm.at[pl.ds(0, _SP)], rows_v.at[b, i], sem_g
                ).wait()

        def fire_write(g, b):
            pltpu.async_copy(
                rows_v.at[b], out_hbm.at[pl.ds(bat_base + g * _BB, _BB)], sem_w,
            )

        def drain_write(g, b):
            pltpu.make_async_copy(
                rows_v.at[b], out_hbm.at[pl.ds(bat_base + g * _BB, _BB)], sem_w,
            ).wait()

        pltpu.async_copy(
            idx_hbm.at[pl.ds(row_base, bat_per_w * _SP)], idx_v, sem_i
        ).wait()

        fire_gathers(0, 0)
        drain_gathers(0)
        fire_write(0, 0)
        fire_gathers(1, 1)

        @pl.loop(0, n_groups - 2, step=2)
        def steady(k):
            for u in range(2):
                g = k + 1 + u
                b = (u + 1) % 2
                nb = 1 - b
                drain_gathers(b)
                fire_write(g, b)
                drain_write(g - 1, nb)
                fire_gathers(g + 1, nb)

        drain_gathers(1)
        fire_write(n_groups - 1, 1)
        drain_write(n_groups - 2, 0)
        drain_write(n_groups - 1, 1)

    return gather


def kernel(actions, table):
    b, s = actions.shape
    idx = jnp.pad(actions.astype(jnp.int32), ((0, 0), (0, _SP - s))).reshape(-1)
    table_p = jnp.pad(table, ((0, 0), (0, _DP - _D)))
    out = _make_gather(b)(table_p, idx)
    return out[:, :s, :_D]


# speedup vs baseline: 1.8975x; 1.8975x over previous
import functools

import jax
import jax.numpy as jnp
from jax import lax
from jax.experimental import pallas as pl
from jax.experimental.pallas import tpu as pltpu
from jax.experimental.pallas import tpu_sc as plsc

_D = 64
_DP = 128
_SP = 56
_NC = 2
_NS = 16
_NW = _NC * _NS
_BB = 8


def _make_gather(n_batch: int):
    assert n_batch % (_NW * _BB) == 0
    bat_per_w = n_batch // _NW
    n_groups = bat_per_w // _BB
    assert n_groups % 2 == 0 and n_groups >= 4
    chunk_rows = _BB * _SP
    mesh = plsc.VectorSubcoreMesh(core_axis_name="c", subcore_axis_name="s")

    @functools.partial(
        pl.kernel,
        out_type=jax.ShapeDtypeStruct((n_batch, _SP, _DP), jnp.float32),
        mesh=mesh,
        scratch_types=[
            pltpu.VMEM((bat_per_w * _SP,), jnp.int32),
            pltpu.VMEM((2, _BB, _SP, _D), jnp.float32),
            pltpu.SemaphoreType.DMA,
            pltpu.SemaphoreType.DMA,
            pltpu.SemaphoreType.DMA,
        ],
        compiler_params=pltpu.CompilerParams(use_tc_tiling_on_sc=False),
    )
    def gather(table_hbm, idx_hbm, out_hbm, idx_v, rows_v, sem_i, sem_g, sem_w):
        wid = lax.axis_index("s") * _NC + lax.axis_index("c")
        row_base = wid * bat_per_w * _SP
        bat_base = wid * bat_per_w

        def fire_gathers(g, b):
            for i in range(_BB):
                pltpu.async_copy(
                    table_hbm.at[idx_v.at[pl.ds(g * chunk_rows + i * _SP, _SP)]],
                    rows_v.at[b, i], sem_g,
                )

        def drain_gathers(b):
            for i in range(_BB):
                pltpu.make_async_copy(
                    table_hbm.at[pl.ds(0, _SP)], rows_v.at[b, i], sem_g
                ).wait()

        def fire_write(g, b):
            pltpu.async_copy(
                rows_v.at[b],
                out_hbm.at[pl.ds(bat_base + g * _BB, _BB), pl.ds(0, _SP),
                           pl.ds(0, _D)],
                sem_w,
            )

        def drain_write(g, b):
            pltpu.make_async_copy(
                rows_v.at[b],
                out_hbm.at[pl.ds(bat_base + g * _BB, _BB), pl.ds(0, _SP),
                           pl.ds(0, _D)],
                sem_w,
            ).wait()

        pltpu.async_copy(
            idx_hbm.at[pl.ds(row_base, bat_per_w * _SP)], idx_v, sem_i
        ).wait()

        fire_gathers(0, 0)
        drain_gathers(0)
        fire_write(0, 0)
        fire_gathers(1, 1)

        @pl.loop(0, n_groups - 2, step=2)
        def steady(k):
            for u in range(2):
                g = k + 1 + u
                b = (u + 1) % 2
                nb = 1 - b
                drain_gathers(b)
                fire_write(g, b)
                drain_write(g - 1, nb)
                fire_gathers(g + 1, nb)

        drain_gathers(1)
        fire_write(n_groups - 1, 1)
        drain_write(n_groups - 2, 0)
        drain_write(n_groups - 1, 1)

    return gather


def kernel(actions, table):
    b, s = actions.shape
    idx = jnp.pad(actions.astype(jnp.int32), ((0, 0), (0, _SP - s))).reshape(-1)
    out = _make_gather(b)(table, idx)
    return out[:, :s, :_D]


# speedup vs baseline: 4.9968x; 2.6333x over previous
import functools

import jax
import jax.numpy as jnp
from jax import lax
from jax.experimental import pallas as pl
from jax.experimental.pallas import tpu as pltpu
from jax.experimental.pallas import tpu_sc as plsc

_D = 64
_NC = 2
_NS = 16
_NW = _NC * _NS
_IDX_PER_STREAM = 128
_KG = 5
_CHUNK = _IDX_PER_STREAM * _KG
_SB = 512


def _make_gather(n_rows: int):
    assert n_rows % (_NW * _CHUNK) == 0
    rows_per_w = n_rows // _NW
    n_groups = rows_per_w // _CHUNK
    assert n_groups % 2 == 0 and n_groups >= 4
    mesh = plsc.VectorSubcoreMesh(core_axis_name="c", subcore_axis_name="s")

    @functools.partial(
        pl.kernel,
        out_type=jax.ShapeDtypeStruct((n_rows, _D), jnp.float32),
        mesh=mesh,
        scratch_types=[
            pltpu.VMEM((rows_per_w,), jnp.int32),
            pltpu.VMEM((2, _CHUNK, _D), jnp.float32),
            pltpu.SemaphoreType.DMA,
            pltpu.SemaphoreType.DMA,
            pltpu.SemaphoreType.DMA,
        ],
        compiler_params=pltpu.CompilerParams(use_tc_tiling_on_sc=False),
    )
    def gather(table_hbm, idx_hbm, out_hbm, idx_v, rows_v, sem_i, sem_g, sem_w):
        wid = lax.axis_index("s") * _NC + lax.axis_index("c")
        base = wid * rows_per_w

        def fire_gathers(g, b):
            for j in range(_KG):
                s = j * _IDX_PER_STREAM
                pltpu.async_copy(
                    table_hbm.at[idx_v.at[pl.ds(g * _CHUNK + s, _IDX_PER_STREAM)]],
                    rows_v.at[b, pl.ds(s, _IDX_PER_STREAM)],
                    sem_g,
                )

        def drain_gathers(b):
            pltpu.make_async_copy(
                table_hbm.at[pl.ds(0, _CHUNK)], rows_v.at[b], sem_g
            ).wait()

        def fire_write(g, b):
            pltpu.async_copy(
                rows_v.at[b], out_hbm.at[pl.ds(base + g * _CHUNK, _CHUNK)], sem_w
            )

        def drain_write(b):
            pltpu.make_async_copy(
                table_hbm.at[pl.ds(0, _CHUNK)], rows_v.at[b], sem_w
            ).wait()

        pltpu.async_copy(idx_hbm.at[pl.ds(base, rows_per_w)], idx_v, sem_i).wait()

        fire_gathers(0, 0)
        drain_gathers(0)
        fire_write(0, 0)
        fire_gathers(1, 1)

        @pl.loop(0, n_groups - 2, step=2)
        def steady(k):
            for u in range(2):
                g = k + 1 + u
                b = (u + 1) % 2
                nb = 1 - b
                drain_gathers(b)
                fire_write(g, b)
                drain_write(nb)
                fire_gathers(g + 1, nb)

        drain_gathers(1)
        fire_write(n_groups - 1, 1)
        drain_write(0)
        drain_write(1)

    return gather


def _transpose_body(xr, outr):
    xx = xr[...]
    outr[0] = xx[:, :_D].T
    outr[1] = xx[:, _D:].T


def _make_transpose(n_batch: int, seq: int):
    assert seq % 2 == 0 and n_batch % _SB == 0
    return pl.pallas_call(
        _transpose_body,
        grid=(seq // 2, n_batch // _SB),
        in_specs=[pl.BlockSpec((_SB, 2 * _D), lambda t, j: (j, t))],
        out_specs=pl.BlockSpec((2, _D, _SB), lambda t, j: (t, 0, j)),
        out_shape=jax.ShapeDtypeStruct((seq, _D, n_batch), jnp.float32),
    )


def kernel(actions, table):
    b, s = actions.shape
    idx = actions.reshape(-1).astype(jnp.int32)
    rows = _make_gather(idx.shape[0])(table, idx)
    out_t = _make_transpose(b, s)(rows.reshape(b, s * _D))
    return out_t.transpose(2, 0, 1)


# speedup vs baseline: 6.7813x; 1.3571x over previous
import functools

import jax
import jax.numpy as jnp
from jax import lax
from jax.experimental import pallas as pl
from jax.experimental.pallas import tpu as pltpu
from jax.experimental.pallas import tpu_sc as plsc

_D = 64
_NC = 2
_NS = 16
_NW = _NC * _NS
_IDX_PER_STREAM = 128
_KG = 5
_CHUNK = _IDX_PER_STREAM * _KG


def _make_gather(n_rows: int):
    assert n_rows % (_NW * _CHUNK) == 0
    rows_per_w = n_rows // _NW
    n_groups = rows_per_w // _CHUNK
    assert n_groups % 2 == 0 and n_groups >= 4
    mesh = plsc.VectorSubcoreMesh(core_axis_name="c", subcore_axis_name="s")

    @functools.partial(
        pl.kernel,
        out_type=jax.ShapeDtypeStruct((n_rows, _D), jnp.float32),
        mesh=mesh,
        scratch_types=[
            pltpu.VMEM((rows_per_w,), jnp.int32),
            pltpu.VMEM((2, _CHUNK, _D), jnp.float32),
            pltpu.SemaphoreType.DMA,
            pltpu.SemaphoreType.DMA,
            pltpu.SemaphoreType.DMA,
        ],
        compiler_params=pltpu.CompilerParams(use_tc_tiling_on_sc=False),
    )
    def gather(table_hbm, idx_hbm, out_hbm, idx_v, rows_v, sem_i, sem_g, sem_w):
        wid = lax.axis_index("s") * _NC + lax.axis_index("c")
        base = wid * rows_per_w

        def fire_gathers(g, b):
            for j in range(_KG):
                s = j * _IDX_PER_STREAM
                pltpu.async_copy(
                    table_hbm.at[idx_v.at[pl.ds(g * _CHUNK + s, _IDX_PER_STREAM)]],
                    rows_v.at[b, pl.ds(s, _IDX_PER_STREAM)],
                    sem_g,
                )

        def drain_gathers(b):
            pltpu.make_async_copy(
                table_hbm.at[pl.ds(0, _CHUNK)], rows_v.at[b], sem_g
            ).wait()

        def fire_write(g, b):
            pltpu.async_copy(
                rows_v.at[b], out_hbm.at[pl.ds(base + g * _CHUNK, _CHUNK)], sem_w
            )

        def drain_write(b):
            pltpu.make_async_copy(
                table_hbm.at[pl.ds(0, _CHUNK)], rows_v.at[b], sem_w
            ).wait()

        pltpu.async_copy(idx_hbm.at[pl.ds(base, rows_per_w)], idx_v, sem_i).wait()

        fire_gathers(0, 0)
        drain_gathers(0)
        fire_write(0, 0)
        fire_gathers(1, 1)

        @pl.loop(0, n_groups - 2, step=2)
        def steady(k):
            for u in range(2):
                g = k + 1 + u
                b = (u + 1) % 2
                nb = 1 - b
                drain_gathers(b)
                fire_write(g, b)
                drain_write(nb)
                fire_gathers(g + 1, nb)

        drain_gathers(1)
        fire_write(n_groups - 1, 1)
        drain_write(0)
        drain_write(1)

    return gather


def kernel(actions, table):
    b, s = actions.shape
    idx = actions.reshape(-1).astype(jnp.int32)
    out = _make_gather(idx.shape[0])(table, idx)
    return out.reshape(b, s, _D)
